# baseline (speedup 1.0000x reference)
import functools

import jax
import jax.numpy as jnp
from jax import lax
from jax.experimental import pallas as pl
from jax.experimental.pallas import tpu as pltpu
from jax.experimental.pallas import tpu_sc as plsc

E = 8
D_MODEL = 768
SEQ = 2048

TOK_TILE = 256
N_TOK_TILES = SEQ // TOK_TILE
ROW_TILE = 256
CAP = 2 * SEQ + E * ROW_TILE
N_ROW_TILES = CAP // ROW_TILE
NW = 32
TPW = SEQ // NW
NVEC = TPW // 16
DV = D_MODEL // 16


def _route_body(x_ref, wg_ref, e1_ref, e2_ref, w1_ref, w2_ref,
                r1_ref, r2_ref, cnt_ref, carry_ref):
    i = pl.program_id(0)
    xt = x_ref[...]
    T = xt.shape[0]

    logits = lax.dot_general(wg_ref[...], xt, (((1,), (1,)), ((), ())),
                             preferred_element_type=jnp.float32)

    iota = lax.broadcasted_iota(jnp.int32, (E, T), 0)
    m1 = jnp.max(logits, axis=0, keepdims=True)
    a1 = jnp.min(jnp.where(logits >= m1, iota, E), axis=0, keepdims=True)
    masked = jnp.where(iota == a1, -jnp.inf, logits)
    m2 = jnp.max(masked, axis=0, keepdims=True)
    a2 = jnp.min(jnp.where(masked >= m2, iota, E), axis=0, keepdims=True)

    w1 = 1.0 / (1.0 + jnp.exp(m2 - m1))
    w2 = 1.0 - w1

    iota16 = lax.broadcasted_iota(jnp.int32, (2 * E, T), 0)
    oh1 = (iota16 == a1).astype(jnp.float32)
    oh2 = (iota16 == a2).astype(jnp.float32)
    oh = oh1 + oh2

    tri = (lax.broadcasted_iota(jnp.int32, (T, T), 0) <
           lax.broadcasted_iota(jnp.int32, (T, T), 1)).astype(jnp.float32)
    csum = lax.dot_general(oh, tri, (((1,), (0,)), ((), ())),
                           preferred_element_type=jnp.float32)

    @pl.when(i == 0)
    def _():
        carry_ref[...] = jnp.zeros_like(carry_ref)

    carry = carry_ref[...]
    r1 = jnp.sum(oh1 * (carry + csum), axis=0, keepdims=True)
    r2 = jnp.sum(oh2 * (carry + csum), axis=0, keepdims=True)
    carry_ref[...] = carry + jnp.sum(oh, axis=1, keepdims=True)

    e1_ref[...] = a1.reshape(1, 1, T)
    e2_ref[...] = a2.reshape(1, 1, T)
    w1_ref[...] = w1.reshape(1, 1, T)
    w2_ref[...] = w2.reshape(1, 1, T)
    r1_ref[...] = r1.astype(jnp.int32).reshape(1, 1, T)
    r2_ref[...] = r2.astype(jnp.int32).reshape(1, 1, T)

    ones = jnp.ones((1, T), jnp.float32)
    ctile = lax.dot_general(ones, oh, (((1,), (1,)), ((), ())),
                            preferred_element_type=jnp.float32)

    @pl.when(i == 0)
    def _():
        cnt_ref[...] = jnp.zeros_like(cnt_ref)

    cnt_ref[...] = cnt_ref[...] + ctile.astype(jnp.int32).reshape(1, 1, 2 * E)


def _route(x2d, Wg):
    T = TOK_TILE
    specs3 = pl.BlockSpec((1, 1, T), lambda i: (i, 0, 0))
    shape3i = jax.ShapeDtypeStruct((N_TOK_TILES, 1, T), jnp.int32)
    shape3f = jax.ShapeDtypeStruct((N_TOK_TILES, 1, T), jnp.float32)
    return pl.pallas_call(
        _route_body,
        grid=(N_TOK_TILES,),
        in_specs=[
            pl.BlockSpec((T, D_MODEL), lambda i: (i, 0)),
            pl.BlockSpec((E, D_MODEL), lambda i: (0, 0)),
        ],
        out_specs=[specs3, specs3, specs3, specs3, specs3, specs3,
                   pl.BlockSpec((1, 1, 2 * E), lambda i: (0, 0, 0))],
        out_shape=[shape3i, shape3i, shape3f, shape3f, shape3i, shape3i,
                   jax.ShapeDtypeStruct((1, 1, 2 * E), jnp.int32)],
        scratch_shapes=[pltpu.VMEM((2 * E, 1), jnp.float32)],
        compiler_params=pltpu.CompilerParams(
            dimension_semantics=("arbitrary",)),
    )(x2d, Wg)


def _dispatch_body(x_hbm, e1_hbm, e2_hbm, r1_hbm, r2_hbm, cnt_hbm,
                   xs_hbm, pos1_hbm, pos2_hbm, eid_hbm,
                   ev1, ev2, rv1, rv2, cnt_v, offp_v, p1_v, p2_v,
                   delta_v, eid_v, xrows, sem):
    wid = lax.axis_index("s") * 2 + lax.axis_index("c")
    base = wid * TPW

    pltpu.sync_copy(e1_hbm.at[pl.ds(base, TPW)], ev1)
    pltpu.sync_copy(e2_hbm.at[pl.ds(base, TPW)], ev2)
    pltpu.sync_copy(r1_hbm.at[pl.ds(base, TPW)], rv1)
    pltpu.sync_copy(r2_hbm.at[pl.ds(base, TPW)], rv2)
    pltpu.sync_copy(cnt_hbm, cnt_v)

    c = cnt_v[...]
    pc = ((c + (ROW_TILE - 1)) >> 8) << 8
    incl = jnp.cumsum(pc)
    offp = incl - pc
    offp_v[...] = offp

    for v in range(NVEC):
        sl = pl.ds(16 * v, 16)
        p1_v[sl] = plsc.load_gather(offp_v, [ev1[sl]]) + rv1[sl]
        p2_v[sl] = plsc.load_gather(offp_v, [ev2[sl]]) + rv2[sl]

    pltpu.sync_copy(p1_v, pos1_hbm.at[pl.ds(base, TPW)])
    pltpu.sync_copy(p2_v, pos2_hbm.at[pl.ds(base, TPW)])

    @pl.when(wid == 0)
    def _():
        zeros = jnp.zeros((16,), jnp.int32)
        delta_v[pl.ds(0, 16)] = zeros
        delta_v[pl.ds(16, 16)] = zeros
        ts = offp >> 8
        lane = lax.broadcasted_iota(jnp.int32, (16,), 0)
        mask = (lane >= 1) & (lane < E)
        plsc.addupdate_scatter(delta_v, [ts], jnp.ones((16,), jnp.int32),
                               mask=mask)
        d0 = delta_v[pl.ds(0, 16)]
        c0 = jnp.cumsum(d0)
        s0 = jnp.sum(d0, axis=0)
        c1 = jnp.cumsum(delta_v[pl.ds(16, 16)]) + s0
        eid_v[pl.ds(0, 16)] = c0
        eid_v[pl.ds(16, 16)] = c1
        pltpu.sync_copy(eid_v, eid_hbm)

    pltpu.sync_copy(x_hbm.at[pl.ds(base, TPW)], xrows)
    pltpu.async_copy(xrows, xs_hbm.at[p1_v], sem).wait()
    pltpu.async_copy(xrows, xs_hbm.at[p2_v], sem).wait()


def _dispatch(x2d, e1, e2, r1, r2, cnt):
    mesh = plsc.VectorSubcoreMesh(core_axis_name="c", subcore_axis_name="s", num_cores=2, num_subcores=16)
    f = pl.kernel(
        _dispatch_body,
        out_type=[
            jax.ShapeDtypeStruct((CAP, D_MODEL), jnp.float32),
            jax.ShapeDtypeStruct((SEQ,), jnp.int32),
            jax.ShapeDtypeStruct((SEQ,), jnp.int32),
            jax.ShapeDtypeStruct((32,), jnp.int32),
        ],
        mesh=mesh,
        scratch_types=[
            pltpu.VMEM((TPW,), jnp.int32),
            pltpu.VMEM((TPW,), jnp.int32),
            pltpu.VMEM((TPW,), jnp.int32),
            pltpu.VMEM((TPW,), jnp.int32),
            pltpu.VMEM((16,), jnp.int32),
            pltpu.VMEM((16,), jnp.int32),
            pltpu.VMEM((TPW,), jnp.int32),
            pltpu.VMEM((TPW,), jnp.int32),
            pltpu.VMEM((32,), jnp.int32),
            pltpu.VMEM((32,), jnp.int32),
            pltpu.VMEM((TPW, D_MODEL), jnp.float32),
            pltpu.SemaphoreType.DMA,
        ],
        compiler_params=pltpu.CompilerParams(needs_layout_passes=False),
    )
    return f(x2d, e1, e2, r1, r2, cnt)


def _expert_mm_body(eid_ref, xs_ref, we_ref, be_ref, ys_ref):
    del eid_ref
    ys_ref[...] = lax.dot_general(
        xs_ref[...], we_ref[0], (((1,), (1,)), ((), ())),
        preferred_element_type=jnp.float32) + be_ref[0]


def _expert_mm(eid, xs, We, be):
    grid_spec = pltpu.PrefetchScalarGridSpec(
        num_scalar_prefetch=1,
        grid=(N_ROW_TILES,),
        in_specs=[
            pl.BlockSpec((ROW_TILE, D_MODEL), lambda i, eid: (i, 0)),
            pl.BlockSpec((1, D_MODEL, D_MODEL), lambda i, eid: (eid[i], 0, 0)),
            pl.BlockSpec((1, 1, D_MODEL), lambda i, eid: (eid[i], 0, 0)),
        ],
        out_specs=pl.BlockSpec((ROW_TILE, D_MODEL), lambda i, eid: (i, 0)),
    )
    return pl.pallas_call(
        _expert_mm_body,
        grid_spec=grid_spec,
        out_shape=jax.ShapeDtypeStruct((CAP, D_MODEL), jnp.float32),
        compiler_params=pltpu.CompilerParams(
            dimension_semantics=("arbitrary",)),
    )(eid, xs, We, be)


def _combine_body(ys_hbm, pos1_hbm, pos2_hbm, w1_hbm, w2_hbm, out_hbm,
                  p1_v, p2_v, w1_v, w2_v, buf1, buf2, sem):
    wid = lax.axis_index("s") * 2 + lax.axis_index("c")
    base = wid * TPW

    pltpu.sync_copy(pos1_hbm.at[pl.ds(base, TPW)], p1_v)
    pltpu.sync_copy(pos2_hbm.at[pl.ds(base, TPW)], p2_v)
    pltpu.sync_copy(w1_hbm.at[pl.ds(base, TPW)], w1_v)
    pltpu.sync_copy(w2_hbm.at[pl.ds(base, TPW)], w2_v)

    pltpu.async_copy(ys_hbm.at[p1_v], buf1, sem).wait()
    pltpu.async_copy(ys_hbm.at[p2_v], buf2, sem).wait()

    def tok_body(t, _):
        idx16 = jnp.full((16,), t, jnp.int32)
        wt1 = plsc.load_gather(w1_v, [idx16])
        wt2 = plsc.load_gather(w2_v, [idx16])
        for j in range(DV):
            sl = pl.ds(16 * j, 16)
            buf1[t, sl] = wt1 * buf1[t, sl] + wt2 * buf2[t, sl]
        return 0

    lax.fori_loop(0, TPW, tok_body, 0)
    pltpu.sync_copy(buf1, out_hbm.at[pl.ds(base, TPW)])


def _combine(ys, pos1, pos2, w1, w2):
    mesh = plsc.VectorSubcoreMesh(core_axis_name="c", subcore_axis_name="s", num_cores=2, num_subcores=16)
    f = pl.kernel(
        _combine_body,
        out_type=jax.ShapeDtypeStruct((SEQ, D_MODEL), jnp.float32),
        mesh=mesh,
        scratch_types=[
            pltpu.VMEM((TPW,), jnp.int32),
            pltpu.VMEM((TPW,), jnp.int32),
            pltpu.VMEM((TPW,), jnp.float32),
            pltpu.VMEM((TPW,), jnp.float32),
            pltpu.VMEM((TPW, D_MODEL), jnp.float32),
            pltpu.VMEM((TPW, D_MODEL), jnp.float32),
            pltpu.SemaphoreType.DMA,
        ],
        compiler_params=pltpu.CompilerParams(needs_layout_passes=False),
    )
    return f(ys, pos1, pos2, w1, w2)


@jax.jit
def _moe(x2d, Wg, We, be):
    e1, e2, w1, w2, r1, r2, cnt = _route(x2d, Wg)
    flat = lambda a: a.reshape(-1)
    xs, pos1, pos2, eid = _dispatch(
        x2d, flat(e1), flat(e2), flat(r1), flat(r2), flat(cnt))
    ys = _expert_mm(eid[:N_ROW_TILES], xs, We, be.reshape(E, 1, D_MODEL))
    return _combine(ys, pos1, pos2, flat(w1), flat(w2))


def kernel(x, Wg, We, be):
    B, S, D = x.shape
    out = _moe(x.reshape(S, D), Wg, We, be)
    return out.reshape(B, S, D)
